# Initial kernel scaffold; baseline (speedup 1.0000x reference)
import jax
import jax.numpy as jnp
from jax.experimental import pallas as pl


def kernel(nodes, edges, receivers, senders, W_message, W_node, mlp_W1, mlp_b1, mlp_W2, mlp_b2, ln_scale, ln_bias):
    raise NotImplementedError("write your pallas kernel here")


# speedup vs baseline: 2.3918x; 2.3918x over previous
import functools

import jax
import jax.numpy as jnp
from jax import lax
from jax.experimental import pallas as pl
from jax.experimental.pallas import tpu as pltpu
from jax.experimental.pallas import tpu_sc as plsc

N_NODES = 10000
N_EDGES = 320000
D_FEAT = 128
D_EDGE = 16

NC = 2
NS = 16
NW = NC * NS
EPT = N_EDGES // NW
CH = 80
NCHUNK = EPT // CH
ACC_ROWS = 10240
RPT = ACC_ROWS // NS


def _sc_segment_sum():
    mesh = plsc.VectorSubcoreMesh(core_axis_name="c", subcore_axis_name="s",
                                  num_cores=NC, num_subcores=NS)

    @functools.partial(
        pl.kernel,
        out_type=jax.ShapeDtypeStruct((NC, ACC_ROWS, D_FEAT), jnp.float32),
        mesh=mesh,
        scratch_types=[
            pltpu.VMEM_SHARED((ACC_ROWS, D_FEAT), jnp.float32),
            pltpu.VMEM((CH,), jnp.int32),
            pltpu.VMEM((CH,), jnp.int32),
            pltpu.VMEM((CH, D_FEAT), jnp.float32),
            pltpu.VMEM((CH, D_FEAT), jnp.float32),
            pltpu.SemaphoreType.DMA,
        ],
    )
    def seg_sum(p_h, m2_h, recv_h, send_h, zn_h, apart_h,
                acc, ridx, sidx, rows, mrows, sem):
        c = lax.axis_index("c")
        s = lax.axis_index("s")
        wid = c * NS + s
        base = wid * EPT
        rslice = pl.ds(s * RPT, RPT)

        pltpu.sync_copy(zn_h.at[rslice], acc.at[rslice])
        plsc.subcore_barrier()

        def chunk(i, carry):
            eb = base + i * CH
            pltpu.sync_copy(send_h.at[pl.ds(eb, CH)], sidx)
            pltpu.sync_copy(recv_h.at[pl.ds(eb, CH)], ridx)
            pltpu.async_copy(p_h.at[sidx], rows, sem).wait()
            pltpu.sync_copy(m2_h.at[pl.ds(eb, CH)], mrows)
            pltpu.sync_copy(rows, acc.at[ridx], add=True)
            pltpu.sync_copy(mrows, acc.at[ridx], add=True)
            return carry

        lax.fori_loop(0, NCHUNK, chunk, 0)
        plsc.subcore_barrier()

        pltpu.sync_copy(acc.at[rslice], apart_h.at[c, rslice])

    return seg_sum


_SEG_SUM = _sc_segment_sum()

BLK = 1000
EBLK = 2000


def _row_block(i):
    return (i, 0)


def _whole(i):
    return (0, 0)


def _mm_body(x_ref, w_ref, o_ref):
    o_ref[...] = jnp.dot(x_ref[...], w_ref[...])


_P_CALL = pl.pallas_call(
    _mm_body,
    grid=(N_NODES // BLK,),
    in_specs=[pl.BlockSpec((BLK, D_FEAT), _row_block),
              pl.BlockSpec((D_FEAT, 128), _whole)],
    out_specs=pl.BlockSpec((BLK, 128), _row_block),
    out_shape=jax.ShapeDtypeStruct((N_NODES, 128), jnp.float32),
)

_M2_CALL = pl.pallas_call(
    _mm_body,
    grid=(N_EDGES // EBLK,),
    in_specs=[pl.BlockSpec((EBLK, D_EDGE), _row_block),
              pl.BlockSpec((D_EDGE, 128), _whole)],
    out_specs=pl.BlockSpec((EBLK, 128), _row_block),
    out_shape=jax.ShapeDtypeStruct((N_EDGES, 128), jnp.float32),
)


def _tc_tail(x_ref, a0_ref, a1_ref,
             w1a_ref, w1b_ref, b1_ref,
             w2_ref, b2_ref, wn_ref, lns_ref, lnb_ref, o_ref):
    x = x_ref[...]
    agg = a0_ref[...] + a1_ref[...]
    h = jnp.maximum(jnp.dot(x, w1a_ref[...]) + jnp.dot(agg, w1b_ref[...])
                    + b1_ref[...], 0.0)
    y = jnp.dot(x, wn_ref[...]) + jnp.dot(h, w2_ref[...]) + b2_ref[...]
    mu = jnp.mean(y, axis=1, keepdims=True)
    var = jnp.mean(jnp.square(y - mu), axis=1, keepdims=True)
    o_ref[...] = (y - mu) * lax.rsqrt(var + 1e-6) * lns_ref[...] + lnb_ref[...]


_TAIL_CALL = pl.pallas_call(
    _tc_tail,
    grid=(N_NODES // BLK,),
    in_specs=[
        pl.BlockSpec((BLK, D_FEAT), _row_block),
        pl.BlockSpec((BLK, 128), _row_block),
        pl.BlockSpec((BLK, 128), _row_block),
        pl.BlockSpec((D_FEAT, 128), _whole),
        pl.BlockSpec((128, 128), _whole),
        pl.BlockSpec((1, 128), _whole),
        pl.BlockSpec((128, 128), _whole),
        pl.BlockSpec((1, 128), _whole),
        pl.BlockSpec((D_FEAT, 128), _whole),
        pl.BlockSpec((1, 128), _whole),
        pl.BlockSpec((1, 128), _whole),
    ],
    out_specs=pl.BlockSpec((BLK, 128), _row_block),
    out_shape=jax.ShapeDtypeStruct((N_NODES, 128), jnp.float32),
)


@jax.jit
def kernel(nodes, edges, receivers, senders, W_message, W_node,
           mlp_W1, mlp_b1, mlp_W2, mlp_b2, ln_scale, ln_bias):
    recv = receivers.astype(jnp.int32)
    send = senders.astype(jnp.int32)
    zn = jnp.zeros((ACC_ROWS, D_FEAT), jnp.float32)

    p = _P_CALL(nodes, W_message[:D_FEAT])
    m2 = _M2_CALL(edges, W_message[D_FEAT:])
    apart = _SEG_SUM(p, m2, recv, send, zn)

    out = _TAIL_CALL(
        nodes, apart[0, :N_NODES], apart[1, :N_NODES],
        mlp_W1[:D_FEAT], mlp_W1[D_FEAT:], mlp_b1.reshape(1, 128),
        mlp_W2, mlp_b2.reshape(1, 128), W_node,
        ln_scale.reshape(1, 128), ln_bias.reshape(1, 128),
    )
    return out


# speedup vs baseline: 3.6367x; 1.5205x over previous
import functools

import jax
import jax.numpy as jnp
from jax import lax
from jax.experimental import pallas as pl
from jax.experimental.pallas import tpu as pltpu
from jax.experimental.pallas import tpu_sc as plsc

N_NODES = 10000
N_EDGES = 320000
D_FEAT = 128
D_EDGE = 16

NC = 2
NS = 16
NW = NC * NS
EPT = N_EDGES // NW
CH = 80
NCHUNK = EPT // CH
ACC_ROWS = 10240
RPT = ACC_ROWS // NS


def _sc_segment_sum():
    mesh = plsc.VectorSubcoreMesh(core_axis_name="c", subcore_axis_name="s",
                                  num_cores=NC, num_subcores=NS)

    @functools.partial(
        pl.kernel,
        out_type=jax.ShapeDtypeStruct((NC, ACC_ROWS, D_FEAT), jnp.float32),
        mesh=mesh,
        scratch_types=[
            pltpu.VMEM_SHARED((ACC_ROWS, D_FEAT), jnp.float32),
            pltpu.VMEM((CH,), jnp.int32),
            pltpu.VMEM((CH,), jnp.int32),
            pltpu.VMEM((CH,), jnp.int32),
            pltpu.VMEM((CH,), jnp.int32),
            pltpu.VMEM((CH, D_FEAT), jnp.float32),
            pltpu.VMEM((CH, D_FEAT), jnp.float32),
            pltpu.VMEM((CH, D_FEAT), jnp.float32),
            pltpu.VMEM((CH, D_FEAT), jnp.float32),
            pltpu.SemaphoreType.DMA,
            pltpu.SemaphoreType.DMA,
            pltpu.SemaphoreType.DMA,
            pltpu.SemaphoreType.DMA,
            pltpu.SemaphoreType.DMA,
            pltpu.SemaphoreType.DMA,
        ],
    )
    def seg_sum(p_h, m2_h, recv_h, send_h, zn_h, apart_h,
                acc, sidx0, sidx1, ridx0, ridx1,
                rows0, rows1, mrows0, mrows1,
                si0, si1, sg0, sg1, sm0, sm1):
        c = lax.axis_index("c")
        s = lax.axis_index("s")
        wid = c * NS + s
        base = wid * EPT
        rslice = pl.ds(s * RPT, RPT)
        sidx = (sidx0, sidx1)
        ridx = (ridx0, ridx1)
        rows = (rows0, rows1)
        mrows = (mrows0, mrows1)
        si = (si0, si1)
        sg = (sg0, sg1)
        sm = (sm0, sm1)

        pltpu.sync_copy(zn_h.at[rslice], acc.at[rslice])
        plsc.subcore_barrier()

        def idxi(i, b):
            @pl.when(i < NCHUNK)
            def _():
                eb = base + i * CH
                pltpu.async_copy(send_h.at[pl.ds(eb, CH)], sidx[b], si[b])
                pltpu.async_copy(recv_h.at[pl.ds(eb, CH)], ridx[b], si[b])

        def idxw(b):
            pltpu.make_async_copy(send_h.at[pl.ds(0, CH)], sidx[b], si[b]).wait()
            pltpu.make_async_copy(recv_h.at[pl.ds(0, CH)], ridx[b], si[b]).wait()

        def datai(i, b):
            pltpu.async_copy(p_h.at[sidx[b]], rows[b], sg[b])
            pltpu.async_copy(m2_h.at[pl.ds(base + i * CH, CH)], mrows[b], sm[b])

        def dataw(b):
            pltpu.make_async_copy(p_h.at[pl.ds(0, CH)], rows[b], sg[b]).wait()
            pltpu.make_async_copy(m2_h.at[pl.ds(0, CH)], mrows[b], sm[b]).wait()

        def scat(b):
            pltpu.sync_copy(rows[b], acc.at[ridx[b]], add=True)
            pltpu.sync_copy(mrows[b], acc.at[ridx[b]], add=True)

        idxi(0, 0)
        idxw(0)
        datai(0, 0)
        idxi(1, 1)

        def step(i, p):
            q = 1 - p
            idxw(q)
            datai(i + 1, q)
            dataw(p)
            scat(p)
            idxi(i + 2, p)

        def pair(j, carry):
            step(2 * j, 0)
            step(2 * j + 1, 1)
            return carry

        lax.fori_loop(0, (NCHUNK - 1) // 2, pair, 0)
        dataw(0)
        scat(0)
        plsc.subcore_barrier()

        pltpu.sync_copy(acc.at[rslice], apart_h.at[c, rslice])

    return seg_sum


_SEG_SUM = _sc_segment_sum()

BLK = 1000
EBLK = 2000


def _row_block(i):
    return (i, 0)


def _whole(i):
    return (0, 0)


def _mm_body(x_ref, w_ref, o_ref):
    o_ref[...] = jnp.dot(x_ref[...], w_ref[...])


_P_CALL = pl.pallas_call(
    _mm_body,
    grid=(N_NODES // BLK,),
    in_specs=[pl.BlockSpec((BLK, D_FEAT), _row_block),
              pl.BlockSpec((D_FEAT, 128), _whole)],
    out_specs=pl.BlockSpec((BLK, 128), _row_block),
    out_shape=jax.ShapeDtypeStruct((N_NODES, 128), jnp.float32),
)

_M2_CALL = pl.pallas_call(
    _mm_body,
    grid=(N_EDGES // EBLK,),
    in_specs=[pl.BlockSpec((EBLK, D_EDGE), _row_block),
              pl.BlockSpec((D_EDGE, 128), _whole)],
    out_specs=pl.BlockSpec((EBLK, 128), _row_block),
    out_shape=jax.ShapeDtypeStruct((N_EDGES, 128), jnp.float32),
)


def _tc_tail(x_ref, a0_ref, a1_ref,
             w1a_ref, w1b_ref, b1_ref,
             w2_ref, b2_ref, wn_ref, lns_ref, lnb_ref, o_ref):
    x = x_ref[...]
    agg = a0_ref[...] + a1_ref[...]
    h = jnp.maximum(jnp.dot(x, w1a_ref[...]) + jnp.dot(agg, w1b_ref[...])
                    + b1_ref[...], 0.0)
    y = jnp.dot(x, wn_ref[...]) + jnp.dot(h, w2_ref[...]) + b2_ref[...]
    mu = jnp.mean(y, axis=1, keepdims=True)
    var = jnp.mean(jnp.square(y - mu), axis=1, keepdims=True)
    o_ref[...] = (y - mu) * lax.rsqrt(var + 1e-6) * lns_ref[...] + lnb_ref[...]


_TAIL_CALL = pl.pallas_call(
    _tc_tail,
    grid=(N_NODES // BLK,),
    in_specs=[
        pl.BlockSpec((BLK, D_FEAT), _row_block),
        pl.BlockSpec((BLK, 128), _row_block),
        pl.BlockSpec((BLK, 128), _row_block),
        pl.BlockSpec((D_FEAT, 128), _whole),
        pl.BlockSpec((128, 128), _whole),
        pl.BlockSpec((1, 128), _whole),
        pl.BlockSpec((128, 128), _whole),
        pl.BlockSpec((1, 128), _whole),
        pl.BlockSpec((D_FEAT, 128), _whole),
        pl.BlockSpec((1, 128), _whole),
        pl.BlockSpec((1, 128), _whole),
    ],
    out_specs=pl.BlockSpec((BLK, 128), _row_block),
    out_shape=jax.ShapeDtypeStruct((N_NODES, 128), jnp.float32),
)


@jax.jit
def kernel(nodes, edges, receivers, senders, W_message, W_node,
           mlp_W1, mlp_b1, mlp_W2, mlp_b2, ln_scale, ln_bias):
    recv = receivers.astype(jnp.int32)
    send = senders.astype(jnp.int32)
    zn = jnp.zeros((ACC_ROWS, D_FEAT), jnp.float32)

    p = _P_CALL(nodes, W_message[:D_FEAT])
    m2 = _M2_CALL(edges, W_message[D_FEAT:])
    apart = _SEG_SUM(p, m2, recv, send, zn)

    out = _TAIL_CALL(
        nodes, apart[0, :N_NODES], apart[1, :N_NODES],
        mlp_W1[:D_FEAT], mlp_W1[D_FEAT:], mlp_b1.reshape(1, 128),
        mlp_W2, mlp_b2.reshape(1, 128), W_node,
        ln_scale.reshape(1, 128), ln_bias.reshape(1, 128),
    )
    return out


# speedup vs baseline: 4.1608x; 1.1441x over previous
import functools

import jax
import jax.numpy as jnp
from jax import lax
from jax.experimental import pallas as pl
from jax.experimental.pallas import tpu as pltpu
from jax.experimental.pallas import tpu_sc as plsc

N_NODES = 10000
N_EDGES = 320000
D_FEAT = 128
D_EDGE = 16

NC = 2
NS = 16
NW = NC * NS
EPT = N_EDGES // NW
CH = 80
NCHUNK = EPT // CH
ACC_ROWS = 10240
RPT = ACC_ROWS // NS


def _sc_pass(gather: bool):
    mesh = plsc.VectorSubcoreMesh(core_axis_name="c", subcore_axis_name="s",
                                  num_cores=NC, num_subcores=NS)

    @functools.partial(
        pl.kernel,
        out_type=jax.ShapeDtypeStruct((NC, ACC_ROWS, D_FEAT), jnp.float32),
        mesh=mesh,
        scratch_types=[
            pltpu.VMEM_SHARED((ACC_ROWS, D_FEAT), jnp.float32),
            pltpu.VMEM((CH,), jnp.int32),
            pltpu.VMEM((CH,), jnp.int32),
            pltpu.VMEM((CH,), jnp.int32),
            pltpu.VMEM((CH,), jnp.int32),
            pltpu.VMEM((CH, D_FEAT), jnp.float32),
            pltpu.VMEM((CH, D_FEAT), jnp.float32),
            pltpu.SemaphoreType.DMA,
            pltpu.SemaphoreType.DMA,
            pltpu.SemaphoreType.DMA,
            pltpu.SemaphoreType.DMA,
        ],
    )
    def seg_sum(data_h, recv_h, send_h, init_h, apart_h,
                acc, sidx0, sidx1, ridx0, ridx1,
                rows0, rows1, si0, si1, sg0, sg1):
        c = lax.axis_index("c")
        s = lax.axis_index("s")
        wid = c * NS + s
        base = wid * EPT
        rslice = pl.ds(s * RPT, RPT)
        sidx = (sidx0, sidx1)
        ridx = (ridx0, ridx1)
        rows = (rows0, rows1)
        si = (si0, si1)
        sg = (sg0, sg1)

        if gather:
            pltpu.sync_copy(init_h.at[rslice], acc.at[rslice])
        else:
            pltpu.sync_copy(init_h.at[c, rslice], acc.at[rslice])
        plsc.subcore_barrier()

        def idxi(i, b):
            @pl.when(i < NCHUNK)
            def _():
                eb = base + i * CH
                if gather:
                    pltpu.async_copy(send_h.at[pl.ds(eb, CH)], sidx[b], si[b])
                pltpu.async_copy(recv_h.at[pl.ds(eb, CH)], ridx[b], si[b])

        def idxw(b):
            if gather:
                pltpu.make_async_copy(send_h.at[pl.ds(0, CH)], sidx[b],
                                      si[b]).wait()
            pltpu.make_async_copy(recv_h.at[pl.ds(0, CH)], ridx[b],
                                  si[b]).wait()

        def datai(i, b):
            if gather:
                pltpu.async_copy(data_h.at[sidx[b]], rows[b], sg[b])
            else:
                pltpu.async_copy(data_h.at[pl.ds(base + i * CH, CH)],
                                 rows[b], sg[b])

        def dataw(b):
            pltpu.make_async_copy(data_h.at[pl.ds(0, CH)], rows[b],
                                  sg[b]).wait()

        def scat(b):
            pltpu.sync_copy(rows[b], acc.at[ridx[b]], add=True)

        idxi(0, 0)
        idxw(0)
        datai(0, 0)
        idxi(1, 1)

        def step(i, p):
            q = 1 - p
            idxw(q)
            datai(i + 1, q)
            dataw(p)
            scat(p)
            idxi(i + 2, p)

        def pair(j, carry):
            step(2 * j, 0)
            step(2 * j + 1, 1)
            return carry

        lax.fori_loop(0, (NCHUNK - 1) // 2, pair, 0)
        dataw(0)
        scat(0)
        plsc.subcore_barrier()

        pltpu.sync_copy(acc.at[rslice], apart_h.at[c, rslice])

    return seg_sum


_SEG_GATHER = _sc_pass(gather=True)
_SEG_LINEAR = _sc_pass(gather=False)

BLK = 1000
EBLK = 2000


def _row_block(i):
    return (i, 0)


def _whole(i):
    return (0, 0)


def _mm_body(x_ref, w_ref, o_ref):
    o_ref[...] = jnp.dot(x_ref[...], w_ref[...])


_P_CALL = pl.pallas_call(
    _mm_body,
    grid=(N_NODES // BLK,),
    in_specs=[pl.BlockSpec((BLK, D_FEAT), _row_block),
              pl.BlockSpec((D_FEAT, 128), _whole)],
    out_specs=pl.BlockSpec((BLK, 128), _row_block),
    out_shape=jax.ShapeDtypeStruct((N_NODES, 128), jnp.float32),
)

_M2_CALL = pl.pallas_call(
    _mm_body,
    grid=(N_EDGES // EBLK,),
    in_specs=[pl.BlockSpec((EBLK, D_EDGE), _row_block),
              pl.BlockSpec((D_EDGE, 128), _whole)],
    out_specs=pl.BlockSpec((EBLK, 128), _row_block),
    out_shape=jax.ShapeDtypeStruct((N_EDGES, 128), jnp.float32),
)


def _tc_tail(x_ref, a0_ref, a1_ref,
             w1a_ref, w1b_ref, b1_ref,
             w2_ref, b2_ref, wn_ref, lns_ref, lnb_ref, o_ref):
    x = x_ref[...]
    agg = a0_ref[0] + a1_ref[0]
    h = jnp.maximum(jnp.dot(x, w1a_ref[...]) + jnp.dot(agg, w1b_ref[...])
                    + b1_ref[...], 0.0)
    y = jnp.dot(x, wn_ref[...]) + jnp.dot(h, w2_ref[...]) + b2_ref[...]
    mu = jnp.mean(y, axis=1, keepdims=True)
    var = jnp.mean(jnp.square(y - mu), axis=1, keepdims=True)
    o_ref[...] = (y - mu) * lax.rsqrt(var + 1e-6) * lns_ref[...] + lnb_ref[...]


_TAIL_CALL = pl.pallas_call(
    _tc_tail,
    grid=(N_NODES // BLK,),
    in_specs=[
        pl.BlockSpec((BLK, D_FEAT), _row_block),
        pl.BlockSpec((1, BLK, 128), lambda i: (0, i, 0)),
        pl.BlockSpec((1, BLK, 128), lambda i: (1, i, 0)),
        pl.BlockSpec((D_FEAT, 128), _whole),
        pl.BlockSpec((128, 128), _whole),
        pl.BlockSpec((1, 128), _whole),
        pl.BlockSpec((128, 128), _whole),
        pl.BlockSpec((1, 128), _whole),
        pl.BlockSpec((D_FEAT, 128), _whole),
        pl.BlockSpec((1, 128), _whole),
        pl.BlockSpec((1, 128), _whole),
    ],
    out_specs=pl.BlockSpec((BLK, 128), _row_block),
    out_shape=jax.ShapeDtypeStruct((N_NODES, 128), jnp.float32),
)


@jax.jit
def kernel(nodes, edges, receivers, senders, W_message, W_node,
           mlp_W1, mlp_b1, mlp_W2, mlp_b2, ln_scale, ln_bias):
    recv = receivers.astype(jnp.int32)
    send = senders.astype(jnp.int32)
    zn = jnp.zeros((ACC_ROWS, D_FEAT), jnp.float32)

    p = _P_CALL(nodes, W_message[:D_FEAT])
    m2 = _M2_CALL(edges, W_message[D_FEAT:])
    a1 = _SEG_GATHER(p, recv, send, zn)
    apart = _SEG_LINEAR(m2, recv, send, a1)

    out = _TAIL_CALL(
        nodes, apart, apart,
        mlp_W1[:D_FEAT], mlp_W1[D_FEAT:], mlp_b1.reshape(1, 128),
        mlp_W2, mlp_b2.reshape(1, 128), W_node,
        ln_scale.reshape(1, 128), ln_bias.reshape(1, 128),
    )
    return out


# speedup vs baseline: 4.1700x; 1.0022x over previous
import functools

import jax
import jax.numpy as jnp
from jax import lax
from jax.experimental import pallas as pl
from jax.experimental.pallas import tpu as pltpu
from jax.experimental.pallas import tpu_sc as plsc

N_NODES = 10000
N_EDGES = 320000
D_FEAT = 128
D_EDGE = 16

NC = 2
NS = 16
NW = NC * NS
EPT = N_EDGES // NW
CH = 80
NCHUNK = EPT // CH
ACC_ROWS = 10240
RPT = ACC_ROWS // NS


def _sc_pass(gather: bool):
    mesh = plsc.VectorSubcoreMesh(core_axis_name="c", subcore_axis_name="s",
                                  num_cores=NC, num_subcores=NS)

    @functools.partial(
        pl.kernel,
        out_type=jax.ShapeDtypeStruct((NC, ACC_ROWS, D_FEAT), jnp.float32),
        mesh=mesh,
        scratch_types=(
            [pltpu.VMEM_SHARED((ACC_ROWS, D_FEAT), jnp.float32)]
            + [pltpu.VMEM((CH,), jnp.int32)] * 8
            + [pltpu.VMEM((CH, D_FEAT), jnp.float32)] * 4
            + [pltpu.SemaphoreType.DMA] * 8
        ),
    )
    def seg_sum(data_h, recv_h, send_h, init_h, apart_h,
                acc, sidx0, sidx1, sidx2, sidx3, ridx0, ridx1, ridx2, ridx3,
                rows0, rows1, rows2, rows3,
                si0, si1, si2, si3, sg0, sg1, sg2, sg3):
        c = lax.axis_index("c")
        s = lax.axis_index("s")
        wid = c * NS + s
        base = wid * EPT
        rslice = pl.ds(s * RPT, RPT)
        sidx = (sidx0, sidx1, sidx2, sidx3)
        ridx = (ridx0, ridx1, ridx2, ridx3)
        rows = (rows0, rows1, rows2, rows3)
        si = (si0, si1, si2, si3)
        sg = (sg0, sg1, sg2, sg3)

        if gather:
            pltpu.sync_copy(init_h.at[rslice], acc.at[rslice])
        else:
            pltpu.sync_copy(init_h.at[c, rslice], acc.at[rslice])
        plsc.subcore_barrier()

        def idxi(i, b):
            @pl.when(i < NCHUNK)
            def _():
                eb = base + i * CH
                if gather:
                    pltpu.async_copy(send_h.at[pl.ds(eb, CH)], sidx[b], si[b])
                pltpu.async_copy(recv_h.at[pl.ds(eb, CH)], ridx[b], si[b])

        def idxw(b):
            if gather:
                pltpu.make_async_copy(send_h.at[pl.ds(0, CH)], sidx[b],
                                      si[b]).wait()
            pltpu.make_async_copy(recv_h.at[pl.ds(0, CH)], ridx[b],
                                  si[b]).wait()

        def datai(i, b):
            if gather:
                pltpu.async_copy(data_h.at[sidx[b]], rows[b], sg[b])
            else:
                pltpu.async_copy(data_h.at[pl.ds(base + i * CH, CH)],
                                 rows[b], sg[b])

        def dataw(b):
            pltpu.make_async_copy(data_h.at[pl.ds(0, CH)], rows[b],
                                  sg[b]).wait()

        def scat(b):
            pltpu.sync_copy(rows[b], acc.at[ridx[b]], add=True)

        idxi(0, 0)
        idxw(0)
        datai(0, 0)
        idxi(1, 1)
        idxw(1)
        datai(1, 1)
        idxi(2, 2)
        idxw(2)
        datai(2, 2)
        idxi(3, 3)

        def step(i, p):
            n = (p + 3) % 4
            idxw(n)
            datai(i + 3, n)
            dataw(p)
            scat(p)
            idxi(i + 4, p)

        def quad(j, carry):
            step(4 * j, 0)
            step(4 * j + 1, 1)
            step(4 * j + 2, 2)
            step(4 * j + 3, 3)
            return carry

        lax.fori_loop(0, 30, quad, 0)
        for i in range(120, NCHUNK):
            p = i % 4
            if i + 3 < NCHUNK:
                n = (p + 3) % 4
                idxw(n)
                datai(i + 3, n)
            dataw(p)
            scat(p)
            if i + 4 < NCHUNK:
                idxi(i + 4, p)
        plsc.subcore_barrier()

        pltpu.sync_copy(acc.at[rslice], apart_h.at[c, rslice])

    return seg_sum


_SEG_GATHER = _sc_pass(gather=True)
_SEG_LINEAR = _sc_pass(gather=False)

BLK = 1000
EBLK = 2000


def _row_block(i):
    return (i, 0)


def _whole(i):
    return (0, 0)


def _mm_body(x_ref, w_ref, o_ref):
    o_ref[...] = jnp.dot(x_ref[...], w_ref[...])


_P_CALL = pl.pallas_call(
    _mm_body,
    grid=(N_NODES // BLK,),
    in_specs=[pl.BlockSpec((BLK, D_FEAT), _row_block),
              pl.BlockSpec((D_FEAT, 128), _whole)],
    out_specs=pl.BlockSpec((BLK, 128), _row_block),
    out_shape=jax.ShapeDtypeStruct((N_NODES, 128), jnp.float32),
)

_M2_CALL = pl.pallas_call(
    _mm_body,
    grid=(N_EDGES // EBLK,),
    in_specs=[pl.BlockSpec((EBLK, D_EDGE), _row_block),
              pl.BlockSpec((D_EDGE, 128), _whole)],
    out_specs=pl.BlockSpec((EBLK, 128), _row_block),
    out_shape=jax.ShapeDtypeStruct((N_EDGES, 128), jnp.float32),
)


def _tc_tail(x_ref, a0_ref, a1_ref,
             w1a_ref, w1b_ref, b1_ref,
             w2_ref, b2_ref, wn_ref, lns_ref, lnb_ref, o_ref):
    x = x_ref[...]
    agg = a0_ref[0] + a1_ref[0]
    h = jnp.maximum(jnp.dot(x, w1a_ref[...]) + jnp.dot(agg, w1b_ref[...])
                    + b1_ref[...], 0.0)
    y = jnp.dot(x, wn_ref[...]) + jnp.dot(h, w2_ref[...]) + b2_ref[...]
    mu = jnp.mean(y, axis=1, keepdims=True)
    var = jnp.mean(jnp.square(y - mu), axis=1, keepdims=True)
    o_ref[...] = (y - mu) * lax.rsqrt(var + 1e-6) * lns_ref[...] + lnb_ref[...]


_TAIL_CALL = pl.pallas_call(
    _tc_tail,
    grid=(N_NODES // BLK,),
    in_specs=[
        pl.BlockSpec((BLK, D_FEAT), _row_block),
        pl.BlockSpec((1, BLK, 128), lambda i: (0, i, 0)),
        pl.BlockSpec((1, BLK, 128), lambda i: (1, i, 0)),
        pl.BlockSpec((D_FEAT, 128), _whole),
        pl.BlockSpec((128, 128), _whole),
        pl.BlockSpec((1, 128), _whole),
        pl.BlockSpec((128, 128), _whole),
        pl.BlockSpec((1, 128), _whole),
        pl.BlockSpec((D_FEAT, 128), _whole),
        pl.BlockSpec((1, 128), _whole),
        pl.BlockSpec((1, 128), _whole),
    ],
    out_specs=pl.BlockSpec((BLK, 128), _row_block),
    out_shape=jax.ShapeDtypeStruct((N_NODES, 128), jnp.float32),
)


@jax.jit
def kernel(nodes, edges, receivers, senders, W_message, W_node,
           mlp_W1, mlp_b1, mlp_W2, mlp_b2, ln_scale, ln_bias):
    recv = receivers.astype(jnp.int32)
    send = senders.astype(jnp.int32)
    zn = jnp.zeros((ACC_ROWS, D_FEAT), jnp.float32)

    p = _P_CALL(nodes, W_message[:D_FEAT])
    m2 = _M2_CALL(edges, W_message[D_FEAT:])
    a1 = _SEG_GATHER(p, recv, send, zn)
    apart = _SEG_LINEAR(m2, recv, send, a1)

    out = _TAIL_CALL(
        nodes, apart, apart,
        mlp_W1[:D_FEAT], mlp_W1[D_FEAT:], mlp_b1.reshape(1, 128),
        mlp_W2, mlp_b2.reshape(1, 128), W_node,
        ln_scale.reshape(1, 128), ln_bias.reshape(1, 128),
    )
    return out
